# baseline (speedup 1.0000x reference)
import jax
import jax.numpy as jnp
from jax.experimental import pallas as pl

B = 4
S = 4096
D_MODEL = 4096
NUM_EXPERTS = 64
NUM_SELECTED = 8

BLK = 512


def _router_body(x_ref, w_ref, probs_ref, gate_ref, idx_ref):
    logits = jax.lax.dot_general(
        w_ref[...], x_ref[...],
        dimension_numbers=(((1,), (1,)), ((), ())),
        preferred_element_type=jnp.float32)
    m = jnp.max(logits, axis=0, keepdims=True)
    e = jnp.exp(logits - m)
    probs = e * (1.0 / jnp.sum(e, axis=0, keepdims=True))
    probs_ref[...] = probs.T

    iota_f = jax.lax.broadcasted_iota(jnp.int32, probs.shape, 0).astype(
        jnp.float32)
    p = probs
    gates = []
    idxs = []
    for k in range(NUM_SELECTED):
        mv = jnp.max(p, axis=0, keepdims=True)
        ic = jnp.min(jnp.where(p == mv, iota_f, float(NUM_EXPERTS)),
                     axis=0, keepdims=True)
        gates.append(mv)
        idxs.append(ic)
        if k + 1 < NUM_SELECTED:
            p = jnp.where(iota_f == ic, -1.0, p)
    g = jnp.concatenate(gates, axis=0)
    i = jnp.concatenate(idxs, axis=0).astype(jnp.int32)
    g = g * (1.0 / jnp.sum(g, axis=0, keepdims=True))
    gate_ref[...] = g.T
    idx_ref[...] = i.T


@jax.jit
def kernel(x, W):
    n = B * S
    xf = x.reshape(n, D_MODEL)

    probs, gate, idx = pl.pallas_call(
        _router_body,
        grid=(n // BLK,),
        in_specs=[
            pl.BlockSpec((BLK, D_MODEL), lambda i: (i, 0)),
            pl.BlockSpec((NUM_EXPERTS, D_MODEL), lambda i: (0, 0)),
        ],
        out_specs=[
            pl.BlockSpec((BLK, NUM_EXPERTS), lambda i: (i, 0)),
            pl.BlockSpec((BLK, NUM_SELECTED), lambda i: (i, 0)),
            pl.BlockSpec((BLK, NUM_SELECTED), lambda i: (i, 0)),
        ],
        out_shape=[
            jax.ShapeDtypeStruct((n, NUM_EXPERTS), jnp.float32),
            jax.ShapeDtypeStruct((n, NUM_SELECTED), jnp.float32),
            jax.ShapeDtypeStruct((n, NUM_SELECTED), jnp.int32),
        ],
    )(xf, W)

    return (
        gate.reshape(B, S, NUM_SELECTED),
        idx.reshape(B, S, NUM_SELECTED),
        probs.reshape(B, S, NUM_EXPERTS),
    )


# speedup vs baseline: 1.0748x; 1.0748x over previous
import jax
import jax.numpy as jnp
from jax.experimental import pallas as pl

B = 4
S = 4096
D_MODEL = 4096
NUM_EXPERTS = 64
NUM_SELECTED = 8

BLK = 1024


def _router_body(x_ref, w_ref, probs_ref, gate_ref, idx_ref):
    logits = jax.lax.dot_general(
        w_ref[...], x_ref[...],
        dimension_numbers=(((1,), (1,)), ((), ())),
        preferred_element_type=jnp.float32)
    m = jnp.max(logits, axis=0, keepdims=True)
    e = jnp.exp(logits - m)
    probs = e * (1.0 / jnp.sum(e, axis=0, keepdims=True))
    probs_ref[...] = probs.T

    iota_f = jax.lax.broadcasted_iota(jnp.int32, probs.shape, 0).astype(
        jnp.float32)
    p = probs
    gates = []
    idxs = []
    for k in range(NUM_SELECTED):
        mv = jnp.max(p, axis=0, keepdims=True)
        ic = jnp.min(jnp.where(p == mv, iota_f, float(NUM_EXPERTS)),
                     axis=0, keepdims=True)
        gates.append(mv)
        idxs.append(ic)
        if k + 1 < NUM_SELECTED:
            p = jnp.where(iota_f == ic, -1.0, p)
    g = jnp.concatenate(gates, axis=0)
    i = jnp.concatenate(idxs, axis=0).astype(jnp.int32)
    g = g * (1.0 / jnp.sum(g, axis=0, keepdims=True))
    gate_ref[...] = g.T
    idx_ref[...] = i.T


@jax.jit
def kernel(x, W):
    n = B * S
    xf = x.reshape(n, D_MODEL)

    probs, gate, idx = pl.pallas_call(
        _router_body,
        grid=(n // BLK,),
        in_specs=[
            pl.BlockSpec((BLK, D_MODEL), lambda i: (i, 0)),
            pl.BlockSpec((NUM_EXPERTS, D_MODEL), lambda i: (0, 0)),
        ],
        out_specs=[
            pl.BlockSpec((BLK, NUM_EXPERTS), lambda i: (i, 0)),
            pl.BlockSpec((BLK, NUM_SELECTED), lambda i: (i, 0)),
            pl.BlockSpec((BLK, NUM_SELECTED), lambda i: (i, 0)),
        ],
        out_shape=[
            jax.ShapeDtypeStruct((n, NUM_EXPERTS), jnp.float32),
            jax.ShapeDtypeStruct((n, NUM_SELECTED), jnp.float32),
            jax.ShapeDtypeStruct((n, NUM_SELECTED), jnp.int32),
        ],
    )(xf, W)

    return (
        gate.reshape(B, S, NUM_SELECTED),
        idx.reshape(B, S, NUM_SELECTED),
        probs.reshape(B, S, NUM_EXPERTS),
    )
